# Initial kernel scaffold; baseline (speedup 1.0000x reference)
import jax
import jax.numpy as jnp
from jax.experimental import pallas as pl


def kernel(graph, r_ij, W_a, W_v, W_d):
    raise NotImplementedError("write your pallas kernel here")


# speedup vs baseline: 39.2523x; 39.2523x over previous
import functools

import jax
import jax.numpy as jnp
from jax import lax
from jax.experimental import pallas as pl
from jax.experimental.pallas import tpu as pltpu
from jax.experimental.pallas import tpu_sc as plsc

NN = 50000
NE = 800000
F = 104
FP = 112
NC = 2
NS = 16
L = 16
NPASS = 2
NCOMBO = NC * NPASS
NOWN = 12504
NLAST = NN - (NCOMBO - 1) * NOWN
NLOC = 12544
ZROWS = NLOC // NS
T15 = NOWN - (NS - 1) * ZROWS
T15L = NLAST - (NS - 1) * ZROWS
B = 80
CH = 2000
NBLK = CH // B
EPT = NE // NS
NCHUNK = EPT // CH

_COS_COEF = (
    1.0, -4.9348021, 4.0587120, -1.3352628, 0.23533063,
    -0.025806880, 1.9295465e-3, -1.0459815e-4, 4.2683787e-6, -1.2196688e-7,
)


def _rsqrt(x):
    i = plsc.bitcast(x, jnp.int32)
    y = plsc.bitcast(jnp.int32(0x5F3759DF) - (i >> 1), jnp.float32)
    for _ in range(3):
        y = y * (1.5 - 0.5 * x * y * y)
    return y


def _cos_pi_u(w):
    acc = jnp.full((L,), _COS_COEF[-1], jnp.float32)
    for c in reversed(_COS_COEF[:-1]):
        acc = acc * w + c
    return acc


def _sc_body(graph_hbm, r_hbm, out_hbm, acc, idx_in, r_in, payload, sidx):
    core = lax.axis_index("c")
    sub = lax.axis_index("s")
    lane = lax.iota(jnp.int32, L)
    zeros16 = jnp.zeros((L,), jnp.float32)

    def pass_body(p, _):
        node_lo = (p * NC + core) * NOWN

        def zrow(i, _):
            for k in range(FP // L):
                payload[i, pl.ds(k * L, L)] = zeros16
            return 0
        lax.fori_loop(0, B, zrow, 0)

        row0 = sub * ZROWS
        for j in range(ZROWS // B):
            pltpu.sync_copy(payload, acc.at[pl.ds(row0 + j * B, B), :])
        rem = ZROWS % B
        if rem:
            pltpu.sync_copy(payload.at[pl.ds(0, rem), :],
                            acc.at[pl.ds(row0 + (ZROWS // B) * B, rem), :])
        plsc.subcore_barrier()

        ebase0 = sub * EPT

        def chunk_body(ci, _):
            off = ebase0 + ci * CH
            pltpu.sync_copy(graph_hbm.at[pl.ds(off, CH)], idx_in)
            pltpu.sync_copy(r_hbm.at[pl.ds(off * 3, CH * 3)], r_in)

            def blk_body(bi, _):
                base = bi * B
                for g in range(B // L):
                    o = base + g * L
                    nodeidx = idx_in[pl.ds(o, L)]
                    local = nodeidx - node_lo
                    ok = (local >= 0) & (local < NOWN)
                    sidx[pl.ds(g * L, L)] = jnp.where(ok, local, NOWN)

                    gidx = o * 3 + lane * 3
                    x = plsc.load_gather(r_in, [gidx])
                    y = plsc.load_gather(r_in, [gidx + 1])
                    z = plsc.load_gather(r_in, [gidx + 2])

                    rsq = x * x + y * y + z * z
                    xsq = rsq * 0.2
                    relu = jnp.maximum(1.0 - xsq, 0.0)
                    u = xsq * _rsqrt(jnp.maximum(xsq, 1e-30))
                    u = jnp.minimum(u, 1.0)
                    c1 = _cos_pi_u(u * u)
                    cs = [jnp.full((L,), 1.0, jnp.float32), c1]
                    for _ in range(6):
                        cs.append(2.0 * c1 * cs[-1] - cs[-2])
                    rad = [relu] + [cn * relu for cn in cs[1:]]

                    inv = _rsqrt(1.0 + 11.56 * rsq)
                    s = 3.4 * inv
                    rq = [x * s, y * s, z * s]

                    erow = lane + g * L
                    for c in range(8):
                        radc = rad[c]
                        plsc.store_scatter(
                            payload, [erow, jnp.full((L,), c, jnp.int32)], radc)
                        rv = [radc * rq[k] for k in range(3)]
                        for k in range(3):
                            plsc.store_scatter(
                                payload,
                                [erow, jnp.full((L,), 8 + c * 3 + k, jnp.int32)],
                                rv[k])
                        for j in range(3):
                            for k in range(3):
                                plsc.store_scatter(
                                    payload,
                                    [erow,
                                     jnp.full((L,), 32 + c * 9 + j * 3 + k,
                                              jnp.int32)],
                                    rv[j] * rq[k])

                pltpu.sync_copy(payload, acc.at[sidx], add=True)
                return 0

            lax.fori_loop(0, NBLK, blk_body, 0)
            return 0

        lax.fori_loop(0, NCHUNK, chunk_body, 0)
        plsc.subcore_barrier()

        r0 = sub * ZROWS
        m = p * NC + core

        @pl.when(sub < NS - 1)
        def _():
            pltpu.sync_copy(acc.at[pl.ds(r0, ZROWS), :],
                            out_hbm.at[pl.ds(node_lo + r0, ZROWS), :])

        @pl.when((sub == NS - 1) & (m < NCOMBO - 1))
        def _():
            pltpu.sync_copy(acc.at[pl.ds(r0, T15), :],
                            out_hbm.at[pl.ds(node_lo + r0, T15), :])

        @pl.when((sub == NS - 1) & (m == NCOMBO - 1))
        def _():
            pltpu.sync_copy(acc.at[pl.ds(r0, T15L), :],
                            out_hbm.at[pl.ds(node_lo + r0, T15L), :])

        plsc.subcore_barrier()
        return 0

    lax.fori_loop(0, NPASS, pass_body, 0)


@functools.lru_cache(maxsize=None)
def _sc_accumulate_fn():
    return functools.partial(
        pl.kernel,
        out_type=jax.ShapeDtypeStruct((NN, FP), jnp.float32),
        mesh=plsc.VectorSubcoreMesh(
            core_axis_name="c", subcore_axis_name="s",
            num_cores=NC, num_subcores=NS),
        scratch_types=[
            pltpu.VMEM_SHARED((NLOC, FP), jnp.float32),
            pltpu.VMEM((CH,), jnp.int32),
            pltpu.VMEM((CH * 3,), jnp.float32),
            pltpu.VMEM((B, FP), jnp.float32),
            pltpu.VMEM((B,), jnp.int32),
        ],
        compiler_params=pltpu.CompilerParams(
            needs_layout_passes=False, use_tc_tiling_on_sc=False),
    )(_sc_body)


BN = 2000


def _tc_body(a_ref, wa_ref, wv_ref, wd_ref, oa_ref, ov_ref, od_ref):
    a = a_ref[...]
    oa_ref[...] = jnp.dot(a[:, 0:8], wa_ref[...],
                          preferred_element_type=jnp.float32)
    ov_ref[...] = jnp.dot(a[:, 8:32], wv_ref[...],
                          preferred_element_type=jnp.float32)
    od_ref[...] = jnp.dot(a[:, 32:104], wd_ref[...],
                          preferred_element_type=jnp.float32)


def _tc_matmul(A, W_a, Wv2, Wd2):
    grid = (NN // BN,)
    return pl.pallas_call(
        _tc_body,
        grid=grid,
        in_specs=[
            pl.BlockSpec((BN, FP), lambda i: (i, 0)),
            pl.BlockSpec((8, 64), lambda i: (0, 0)),
            pl.BlockSpec((24, 192), lambda i: (0, 0)),
            pl.BlockSpec((72, 576), lambda i: (0, 0)),
        ],
        out_specs=[
            pl.BlockSpec((BN, 64), lambda i: (i, 0)),
            pl.BlockSpec((BN, 192), lambda i: (i, 0)),
            pl.BlockSpec((BN, 576), lambda i: (i, 0)),
        ],
        out_shape=[
            jax.ShapeDtypeStruct((NN, 64), jnp.float32),
            jax.ShapeDtypeStruct((NN, 192), jnp.float32),
            jax.ShapeDtypeStruct((NN, 576), jnp.float32),
        ],
    )(A, W_a, Wv2, Wd2)


def kernel(graph, r_ij, W_a, W_v, W_d):
    A = _sc_accumulate_fn()(graph, r_ij.reshape(-1))
    eye3 = jnp.eye(3, dtype=jnp.float32)
    eye9 = jnp.eye(9, dtype=jnp.float32)
    Wv2 = jnp.einsum('cd,kl->ckdl', W_v, eye3).reshape(24, 192)
    Wd2 = jnp.einsum('cd,jl->cjdl', W_d, eye9).reshape(72, 576)
    B_a, B_v, B_d = _tc_matmul(A, W_a, Wv2, Wd2)
    return (B_a, B_v.reshape(NN, 64, 3), B_d.reshape(NN, 64, 3, 3))


# speedup vs baseline: 49.8527x; 1.2701x over previous
import functools

import jax
import jax.numpy as jnp
from jax import lax
from jax.experimental import pallas as pl
from jax.experimental.pallas import tpu as pltpu
from jax.experimental.pallas import tpu_sc as plsc

NN = 50000
NE = 800000
F = 104
FP = 112
NC = 2
NS = 16
L = 16
NPASS = 2
NCOMBO = NC * NPASS
NOWN = 12504
NLAST = NN - (NCOMBO - 1) * NOWN
NLOC = 12544
ZROWS = NLOC // NS
T15 = NOWN - (NS - 1) * ZROWS
T15L = NLAST - (NS - 1) * ZROWS
B = 80
CH = 2000
NBLK = CH // B
EPT = NE // NS
NCHUNK = EPT // CH

_COS_COEF = (
    1.0, -4.9348021, 4.0587120, -1.3352628, 0.23533063,
    -0.025806880, 1.9295465e-3, -1.0459815e-4, 4.2683787e-6, -1.2196688e-7,
)


def _rsqrt(x):
    i = plsc.bitcast(x, jnp.int32)
    y = plsc.bitcast(jnp.int32(0x5F3759DF) - (i >> 1), jnp.float32)
    for _ in range(3):
        y = y * (1.5 - 0.5 * x * y * y)
    return y


def _cos_pi_u(w):
    acc = jnp.full((L,), _COS_COEF[-1], jnp.float32)
    for c in reversed(_COS_COEF[:-1]):
        acc = acc * w + c
    return acc


def _sc_body(graph_hbm, r_hbm, out_hbm, acc, idx_in, r_in, payload, sidx, wl):
    core = lax.axis_index("c")
    sub = lax.axis_index("s")
    lane = lax.iota(jnp.int32, L)
    zeros16 = jnp.zeros((L,), jnp.float32)

    idx_in[pl.ds(CH, L)] = jnp.full((L,), -1, jnp.int32)

    def pass_body(p, _):
        node_lo = (p * NC + core) * NOWN

        def zrow(i, _):
            for k in range(FP // L):
                payload[i, pl.ds(k * L, L)] = zeros16
            return 0
        lax.fori_loop(0, B, zrow, 0)

        row0 = sub * ZROWS
        for j in range(ZROWS // B):
            pltpu.sync_copy(payload, acc.at[pl.ds(row0 + j * B, B), :])
        rem = ZROWS % B
        if rem:
            pltpu.sync_copy(payload.at[pl.ds(0, rem), :],
                            acc.at[pl.ds(row0 + (ZROWS // B) * B, rem), :])
        plsc.subcore_barrier()

        ebase0 = sub * EPT

        def chunk_body(ci, _):
            off = ebase0 + ci * CH
            pltpu.sync_copy(graph_hbm.at[pl.ds(off, CH)], idx_in.at[pl.ds(0, CH)])
            pltpu.sync_copy(r_hbm.at[pl.ds(off * 3, CH * 3)], r_in.at[pl.ds(0, CH * 3)])

            def scan_g(gi, cnt):
                nodeidx = idx_in[pl.ds(gi * L, L)]
                local = nodeidx - node_lo
                ok = (local >= 0) & (local < NOWN)
                plsc.store_compressed(wl.at[pl.ds(cnt, L)], lane + gi * L,
                                      mask=ok)
                return cnt + jnp.max(plsc.all_reduce_population_count(ok))

            cnt = lax.fori_loop(0, CH // L, scan_g, 0)

            for t in range(B // L):
                wl[pl.ds(cnt + t * L, L)] = jnp.full((L,), CH, jnp.int32)

            def blk_body(bi, _):
                base = bi * B
                for g in range(B // L):
                    pos = wl[pl.ds(base + g * L, L)]
                    nodeidx = plsc.load_gather(idx_in, [pos])
                    local = nodeidx - node_lo
                    ok = (local >= 0) & (local < NOWN)
                    sidx[pl.ds(g * L, L)] = jnp.where(ok, local, NOWN)

                    gidx = pos * 3
                    x = plsc.load_gather(r_in, [gidx])
                    y = plsc.load_gather(r_in, [gidx + 1])
                    z = plsc.load_gather(r_in, [gidx + 2])

                    rsq = x * x + y * y + z * z
                    xsq = rsq * 0.2
                    relu = jnp.maximum(1.0 - xsq, 0.0)
                    u = xsq * _rsqrt(jnp.maximum(xsq, 1e-30))
                    u = jnp.minimum(u, 1.0)
                    c1 = _cos_pi_u(u * u)
                    cs = [jnp.full((L,), 1.0, jnp.float32), c1]
                    for _ in range(6):
                        cs.append(2.0 * c1 * cs[-1] - cs[-2])
                    rad = [relu] + [cn * relu for cn in cs[1:]]

                    inv = _rsqrt(1.0 + 11.56 * rsq)
                    s = 3.4 * inv
                    rq = [x * s, y * s, z * s]

                    erow = lane + g * L
                    for c in range(8):
                        radc = rad[c]
                        plsc.store_scatter(
                            payload, [erow, jnp.full((L,), c, jnp.int32)], radc)
                        rv = [radc * rq[k] for k in range(3)]
                        for k in range(3):
                            plsc.store_scatter(
                                payload,
                                [erow, jnp.full((L,), 8 + c * 3 + k, jnp.int32)],
                                rv[k])
                        for j in range(3):
                            for k in range(3):
                                plsc.store_scatter(
                                    payload,
                                    [erow,
                                     jnp.full((L,), 32 + c * 9 + j * 3 + k,
                                              jnp.int32)],
                                    rv[j] * rq[k])

                pltpu.sync_copy(payload, acc.at[sidx], add=True)
                return 0

            lax.fori_loop(0, (cnt + B - 1) // B, blk_body, 0)
            return 0

        lax.fori_loop(0, NCHUNK, chunk_body, 0)
        plsc.subcore_barrier()

        r0 = sub * ZROWS
        m = p * NC + core

        @pl.when(sub < NS - 1)
        def _():
            pltpu.sync_copy(acc.at[pl.ds(r0, ZROWS), :],
                            out_hbm.at[pl.ds(node_lo + r0, ZROWS), :])

        @pl.when((sub == NS - 1) & (m < NCOMBO - 1))
        def _():
            pltpu.sync_copy(acc.at[pl.ds(r0, T15), :],
                            out_hbm.at[pl.ds(node_lo + r0, T15), :])

        @pl.when((sub == NS - 1) & (m == NCOMBO - 1))
        def _():
            pltpu.sync_copy(acc.at[pl.ds(r0, T15L), :],
                            out_hbm.at[pl.ds(node_lo + r0, T15L), :])

        plsc.subcore_barrier()
        return 0

    lax.fori_loop(0, NPASS, pass_body, 0)


@functools.lru_cache(maxsize=None)
def _sc_accumulate_fn():
    return functools.partial(
        pl.kernel,
        out_type=jax.ShapeDtypeStruct((NN, FP), jnp.float32),
        mesh=plsc.VectorSubcoreMesh(
            core_axis_name="c", subcore_axis_name="s",
            num_cores=NC, num_subcores=NS),
        scratch_types=[
            pltpu.VMEM_SHARED((NLOC, FP), jnp.float32),
            pltpu.VMEM((CH + L,), jnp.int32),
            pltpu.VMEM((CH * 3 + 3 * L,), jnp.float32),
            pltpu.VMEM((B, FP), jnp.float32),
            pltpu.VMEM((B,), jnp.int32),
            pltpu.VMEM((CH + B,), jnp.int32),
        ],
        compiler_params=pltpu.CompilerParams(
            needs_layout_passes=False, use_tc_tiling_on_sc=False),
    )(_sc_body)


BN = 2000


def _tc_body(a_ref, wa_ref, wv_ref, wd_ref, oa_ref, ov_ref, od_ref):
    a = a_ref[...]
    oa_ref[...] = jnp.dot(a[:, 0:8], wa_ref[...],
                          preferred_element_type=jnp.float32)
    ov_ref[...] = jnp.dot(a[:, 8:32], wv_ref[...],
                          preferred_element_type=jnp.float32)
    od_ref[...] = jnp.dot(a[:, 32:104], wd_ref[...],
                          preferred_element_type=jnp.float32)


def _tc_matmul(A, W_a, Wv2, Wd2):
    grid = (NN // BN,)
    return pl.pallas_call(
        _tc_body,
        grid=grid,
        in_specs=[
            pl.BlockSpec((BN, FP), lambda i: (i, 0)),
            pl.BlockSpec((8, 64), lambda i: (0, 0)),
            pl.BlockSpec((24, 192), lambda i: (0, 0)),
            pl.BlockSpec((72, 576), lambda i: (0, 0)),
        ],
        out_specs=[
            pl.BlockSpec((BN, 64), lambda i: (i, 0)),
            pl.BlockSpec((BN, 192), lambda i: (i, 0)),
            pl.BlockSpec((BN, 576), lambda i: (i, 0)),
        ],
        out_shape=[
            jax.ShapeDtypeStruct((NN, 64), jnp.float32),
            jax.ShapeDtypeStruct((NN, 192), jnp.float32),
            jax.ShapeDtypeStruct((NN, 576), jnp.float32),
        ],
    )(A, W_a, Wv2, Wd2)


def kernel(graph, r_ij, W_a, W_v, W_d):
    A = _sc_accumulate_fn()(graph, r_ij.reshape(-1))
    eye3 = jnp.eye(3, dtype=jnp.float32)
    eye9 = jnp.eye(9, dtype=jnp.float32)
    Wv2 = jnp.einsum('cd,kl->ckdl', W_v, eye3).reshape(24, 192)
    Wd2 = jnp.einsum('cd,jl->cjdl', W_d, eye9).reshape(72, 576)
    B_a, B_v, B_d = _tc_matmul(A, W_a, Wv2, Wd2)
    return (B_a, B_v.reshape(NN, 64, 3), B_d.reshape(NN, 64, 3, 3))


# speedup vs baseline: 52.4479x; 1.0521x over previous
import functools

import numpy as np
import jax
import jax.numpy as jnp
from jax import lax
from jax.experimental import pallas as pl
from jax.experimental.pallas import tpu as pltpu
from jax.experimental.pallas import tpu_sc as plsc

NN = 50000
NE = 800000
FP = 80
NC = 2
NS = 16
L = 16
NPASS = 2
NCOMBO = NC * NPASS
NOWN = 12504
NLAST = NN - (NCOMBO - 1) * NOWN
NLOC = 12544
ZROWS = NLOC // NS
T15 = NOWN - (NS - 1) * ZROWS
T15L = NLAST - (NS - 1) * ZROWS
B = 80
CH = 10000
EPT = NE // NS
NCHUNK = EPT // CH

_COS_COEF = (
    1.0, -4.9348021, 4.0587120, -1.3352628, 0.23533063,
    -0.025806880, 1.9295465e-3, -1.0459815e-4, 4.2683787e-6, -1.2196688e-7,
)


def _rsqrt(x):
    i = plsc.bitcast(x, jnp.int32)
    y = plsc.bitcast(jnp.int32(0x5F3759DF) - (i >> 1), jnp.float32)
    for _ in range(3):
        y = y * (1.5 - 0.5 * x * y * y)
    return y


def _cos_pi_u(w):
    acc = jnp.full((L,), _COS_COEF[-1], jnp.float32)
    for c in reversed(_COS_COEF[:-1]):
        acc = acc * w + c
    return acc


def _sc_body(graph_hbm, r_hbm, out_hbm, acc, idx_in, r_in, payload, sidx,
             wl, sem):
    core = lax.axis_index("c")
    sub = lax.axis_index("s")
    lane = lax.iota(jnp.int32, L)
    zeros16 = jnp.zeros((L,), jnp.float32)

    idx_in[pl.ds(CH, L)] = jnp.full((L,), -1, jnp.int32)

    def pass_body(p, _):
        node_lo = (p * NC + core) * NOWN
        m = p * NC + core

        def zrow(i, _):
            for k in range(FP // L):
                payload[0, i, pl.ds(k * L, L)] = zeros16
            return 0
        lax.fori_loop(0, B, zrow, 0)

        row0 = sub * ZROWS

        def zcp(j, _):
            pltpu.sync_copy(payload.at[0], acc.at[pl.ds(row0 + j * B, B), :])
            return 0
        lax.fori_loop(0, ZROWS // B, zcp, 0)
        rem = ZROWS % B
        if rem:
            pltpu.sync_copy(payload.at[0].at[pl.ds(0, rem), :],
                            acc.at[pl.ds(row0 + (ZROWS // B) * B, rem), :])
        plsc.subcore_barrier()

        ebase0 = sub * EPT

        def chunk_body(ci, _):
            off = ebase0 + ci * CH
            pltpu.sync_copy(graph_hbm.at[pl.ds(off, CH)],
                            idx_in.at[pl.ds(0, CH)])
            pltpu.sync_copy(r_hbm.at[pl.ds(off * 3, CH * 3)],
                            r_in.at[pl.ds(0, CH * 3)])

            def scan_g(gi, cnt):
                nodeidx = idx_in[pl.ds(gi * L, L)]
                local = nodeidx - node_lo
                ok = (local >= 0) & (local < NOWN)
                plsc.store_compressed(wl.at[pl.ds(cnt, L)], lane + gi * L,
                                      mask=ok)
                return cnt + jnp.max(plsc.all_reduce_population_count(ok))

            cnt = lax.fori_loop(0, CH // L, scan_g, 0)

            for t in range(B // L):
                wl[pl.ds(cnt + t * L, L)] = jnp.full((L,), CH, jnp.int32)

            nblk = (cnt + B - 1) // B

            def blk_body(bi, _):
                slot = lax.rem(bi, 2)
                slotv = jnp.zeros((L,), jnp.int32) + slot

                @pl.when(bi >= 2)
                def _():
                    pltpu.make_async_copy(payload.at[slot],
                                          acc.at[sidx.at[slot]], sem).wait()

                base = bi * B
                for g in range(B // L):
                    pos = wl[pl.ds(base + g * L, L)]
                    nodeidx = plsc.load_gather(idx_in, [pos])
                    local = nodeidx - node_lo
                    ok = (local >= 0) & (local < NOWN)
                    plsc.store_scatter(sidx, [slotv, lane + g * L],
                                       jnp.where(ok, local, NOWN))

                    gidx = pos * 3
                    x = plsc.load_gather(r_in, [gidx])
                    y = plsc.load_gather(r_in, [gidx + 1])
                    z = plsc.load_gather(r_in, [gidx + 2])

                    rsq = x * x + y * y + z * z
                    xsq = rsq * 0.2
                    relu = jnp.maximum(1.0 - xsq, 0.0)
                    u = xsq * _rsqrt(jnp.maximum(xsq, 1e-30))
                    u = jnp.minimum(u, 1.0)
                    c1 = _cos_pi_u(u * u)
                    cs = [jnp.full((L,), 1.0, jnp.float32), c1]
                    for _ in range(6):
                        cs.append(2.0 * c1 * cs[-1] - cs[-2])
                    rad = [relu] + [cn * relu for cn in cs[1:]]

                    inv = _rsqrt(1.0 + 11.56 * rsq)
                    s = 3.4 * inv
                    rq = [x * s, y * s, z * s]
                    q6 = [rq[0] * rq[0], rq[0] * rq[1], rq[0] * rq[2],
                          rq[1] * rq[1], rq[1] * rq[2], rq[2] * rq[2]]

                    erow = lane + g * L
                    for c in range(8):
                        radc = rad[c]
                        plsc.store_scatter(
                            payload,
                            [slotv, erow, jnp.full((L,), c, jnp.int32)],
                            radc)
                        for k in range(3):
                            plsc.store_scatter(
                                payload,
                                [slotv, erow,
                                 jnp.full((L,), 8 + c * 3 + k, jnp.int32)],
                                radc * rq[k])
                        for uu in range(6):
                            plsc.store_scatter(
                                payload,
                                [slotv, erow,
                                 jnp.full((L,), 32 + c * 6 + uu, jnp.int32)],
                                radc * q6[uu])

                pltpu.async_copy(payload.at[slot], acc.at[sidx.at[slot]],
                                 sem, add=True)
                return 0

            lax.fori_loop(0, nblk, blk_body, 0)

            @pl.when(nblk >= 2)
            def _():
                s2 = lax.rem(nblk, 2)
                pltpu.make_async_copy(payload.at[s2], acc.at[sidx.at[s2]],
                                      sem).wait()

            @pl.when(nblk >= 1)
            def _():
                s1 = lax.rem(nblk - 1, 2)
                pltpu.make_async_copy(payload.at[s1], acc.at[sidx.at[s1]],
                                      sem).wait()

            return 0

        lax.fori_loop(0, NCHUNK, chunk_body, 0)
        plsc.subcore_barrier()

        r0 = sub * ZROWS

        @pl.when(sub < NS - 1)
        def _():
            pltpu.sync_copy(acc.at[pl.ds(r0, ZROWS), :],
                            out_hbm.at[pl.ds(node_lo + r0, ZROWS), :])

        @pl.when((sub == NS - 1) & (m < NCOMBO - 1))
        def _():
            pltpu.sync_copy(acc.at[pl.ds(r0, T15), :],
                            out_hbm.at[pl.ds(node_lo + r0, T15), :])

        @pl.when((sub == NS - 1) & (m == NCOMBO - 1))
        def _():
            pltpu.sync_copy(acc.at[pl.ds(r0, T15L), :],
                            out_hbm.at[pl.ds(node_lo + r0, T15L), :])

        plsc.subcore_barrier()
        return 0

    lax.fori_loop(0, NPASS, pass_body, 0)


@functools.lru_cache(maxsize=None)
def _sc_accumulate_fn():
    return functools.partial(
        pl.kernel,
        out_type=jax.ShapeDtypeStruct((NN, FP), jnp.float32),
        mesh=plsc.VectorSubcoreMesh(
            core_axis_name="c", subcore_axis_name="s",
            num_cores=NC, num_subcores=NS),
        scratch_types=[
            pltpu.VMEM_SHARED((NLOC, FP), jnp.float32),
            pltpu.VMEM((CH + L,), jnp.int32),
            pltpu.VMEM((CH * 3 + 3 * L,), jnp.float32),
            pltpu.VMEM((2, B, FP), jnp.float32),
            pltpu.VMEM((2, B), jnp.int32),
            pltpu.VMEM((CH + B,), jnp.int32),
            pltpu.SemaphoreType.DMA,
        ],
        compiler_params=pltpu.CompilerParams(
            needs_layout_passes=False, use_tc_tiling_on_sc=False),
    )(_sc_body)


BN = 2000


def _tc_body(a_ref, wa_ref, wv_ref, wd_ref, oa_ref, ov_ref, od_ref):
    a = a_ref[...]
    oa_ref[...] = jnp.dot(a[:, 0:8], wa_ref[...],
                          preferred_element_type=jnp.float32)
    ov_ref[...] = jnp.dot(a[:, 8:32], wv_ref[...],
                          preferred_element_type=jnp.float32)
    od_ref[...] = jnp.dot(a[:, 32:80], wd_ref[...],
                          preferred_element_type=jnp.float32)


def _tc_matmul(A, W_a, Wv2, Wd2):
    grid = (NN // BN,)
    return pl.pallas_call(
        _tc_body,
        grid=grid,
        in_specs=[
            pl.BlockSpec((BN, FP), lambda i: (i, 0)),
            pl.BlockSpec((8, 64), lambda i: (0, 0)),
            pl.BlockSpec((24, 192), lambda i: (0, 0)),
            pl.BlockSpec((48, 576), lambda i: (0, 0)),
        ],
        out_specs=[
            pl.BlockSpec((BN, 64), lambda i: (i, 0)),
            pl.BlockSpec((BN, 192), lambda i: (i, 0)),
            pl.BlockSpec((BN, 576), lambda i: (i, 0)),
        ],
        out_shape=[
            jax.ShapeDtypeStruct((NN, 64), jnp.float32),
            jax.ShapeDtypeStruct((NN, 192), jnp.float32),
            jax.ShapeDtypeStruct((NN, 576), jnp.float32),
        ],
    )(A, W_a, Wv2, Wd2)


_SYM = np.zeros((6, 9), np.float32)
for _u, (_j, _k) in enumerate([(0, 0), (0, 1), (0, 2),
                               (1, 1), (1, 2), (2, 2)]):
    _SYM[_u, _j * 3 + _k] = 1.0
    _SYM[_u, _k * 3 + _j] = 1.0


def kernel(graph, r_ij, W_a, W_v, W_d):
    A = _sc_accumulate_fn()(graph, r_ij.reshape(-1))
    eye3 = jnp.eye(3, dtype=jnp.float32)
    Wv2 = jnp.einsum('cd,kl->ckdl', W_v, eye3).reshape(24, 192)
    Wd2 = jnp.einsum('cd,um->cudm', W_d, jnp.asarray(_SYM)).reshape(48, 576)
    B_a, B_v, B_d = _tc_matmul(A, W_a, Wv2, Wd2)
    return (B_a, B_v.reshape(NN, 64, 3), B_d.reshape(NN, 64, 3, 3))
